# baseline (speedup 1.0000x reference)
import functools

import jax
import jax.numpy as jnp
from jax import lax
from jax.experimental import pallas as pl
from jax.experimental.pallas import tpu as pltpu
from jax.experimental.pallas import tpu_sc as plsc

N_ROWS = 6_400_000
_INFO = plsc.get_sparse_core_info()
_NC = _INFO.num_cores
_NS = _INFO.num_subcores
_NW = _NC * _NS
ROWS_PER_W = N_ROWS // _NW
CHUNK = 2_000
N_CHUNKS = ROWS_PER_W // CHUNK
GROUPS = CHUNK // 16

_mesh = plsc.VectorSubcoreMesh(core_axis_name="c", subcore_axis_name="s")


@functools.partial(
    pl.kernel,
    mesh=_mesh,
    out_type=jax.ShapeDtypeStruct((N_ROWS * 2,), jnp.float32),
    scratch_types=[
        pltpu.VMEM((CHUNK * 5,), jnp.float32),
        pltpu.VMEM((CHUNK * 2,), jnp.float32),
    ],
    compiler_params=pltpu.CompilerParams(needs_layout_passes=False),
)
def _segment_prod_sc(x_hbm, out_hbm, in_v, out_v):
    wid = lax.axis_index("s") * _NC + lax.axis_index("c")

    lanes = lax.iota(jnp.int32, 16)
    in_lanes = lanes * 5
    out_lanes = lanes * 2

    def chunk_body(it, _):
        start = (wid * ROWS_PER_W + it * CHUNK)
        pltpu.sync_copy(x_hbm.at[pl.ds(start * 5, CHUNK * 5)], in_v)

        def group_body(g, _):
            idx = g * 80 + in_lanes
            a = plsc.load_gather(in_v, [idx])
            b = plsc.load_gather(in_v, [idx + 1])
            c = plsc.load_gather(in_v, [idx + 2])
            d = plsc.load_gather(in_v, [idx + 3])
            e = plsc.load_gather(in_v, [idx + 4])
            oidx = g * 32 + out_lanes
            plsc.store_scatter(out_v, [oidx], a * b * c)
            plsc.store_scatter(out_v, [oidx + 1], d * e)
            return 0

        lax.fori_loop(0, GROUPS, group_body, 0)
        pltpu.sync_copy(out_v, out_hbm.at[pl.ds(start * 2, CHUNK * 2)])
        return 0

    lax.fori_loop(0, N_CHUNKS, chunk_body, 0)


def kernel(inputs):
    flat = _segment_prod_sc(inputs.reshape(-1))
    return flat.reshape(N_ROWS, 2)
